# baseline (speedup 1.0000x reference)
import functools

import jax
import jax.numpy as jnp
from jax import lax
from jax.experimental import pallas as pl
from jax.experimental.pallas import tpu as pltpu

C1 = 64
C2 = 32
KH = 3
NPH = 18
NPHP = 32


def _make_conv_kernel(bb, T, TP):
    K1 = KH * C1

    def body(xph_ref, w16_ref, b2_ref, w2c_ref, wih_ref, gbt_ref,
             out_ref, h1_ref, feat_ref):
        lane = lax.broadcasted_iota(jnp.int32, (C1, TP), 1)
        b2 = b2_ref[...]
        gbt = gbt_ref[...]
        w2c = w2c_ref[...]
        wih = wih_ref[...]
        zcol = jnp.zeros((C1, 1), jnp.bfloat16)

        for b in range(bb):
            for kh in range(KH):
                ph = []
                for d in range(4):
                    rd = jnp.dot(w16_ref[256 * d:256 * d + 256, :],
                                 xph_ref[b, kh],
                                 preferred_element_type=jnp.float32)
                    pp = rd.reshape(4, C1, TP)
                    ph.append((jnp.maximum(jnp.maximum(pp[0], pp[1]),
                                           jnp.maximum(pp[2], pp[3]))
                               ).astype(jnp.bfloat16))
                p0 = jnp.where(lane < T, ph[0], 0.0)
                row = kh * C1
                col = b * TP
                h1_ref[1 * K1 + row:1 * K1 + row + C1, col:col + TP] = p0
                h1_ref[2 * K1 + row:2 * K1 + row + C1, col:col + TP] = ph[1]
                h1_ref[3 * K1 + row:3 * K1 + row + C1, col:col + TP] = ph[2]
                h1_ref[4 * K1 + row:4 * K1 + row + C1, col:col + TP] = ph[3]
                h1_ref[0 * K1 + row:0 * K1 + row + C1,
                       col:col + TP] = jnp.concatenate(
                    [zcol, ph[3][:, :TP - 1]], axis=1)
                h1_ref[5 * K1 + row:5 * K1 + row + C1,
                       col:col + TP] = jnp.concatenate(
                    [p0[:, 1:], zcol], axis=1)

        NC = 2 * TP
        for c in range(bb * TP // NC):
            s_c = jnp.dot(w2c, h1_ref[:, c * NC:(c + 1) * NC],
                          preferred_element_type=jnp.float32)
            sc = s_c.reshape(4, C2, NC)
            feat_ref[:, c * NC:(c + 1) * NC] = (
                jnp.maximum(jnp.maximum(sc[0], sc[1]),
                            jnp.maximum(sc[2], sc[3])) + b2
            ).astype(jnp.bfloat16)

        proj = lax.dot_general(
            wih, feat_ref[...], dimension_numbers=(((0,), (0,)), ((), ())),
            preferred_element_type=jnp.float32)
        for b in range(bb):
            out_ref[b] = proj[:, b * TP:(b + 1) * TP] + gbt

    return body


def _make_scan_kernel(T):
    def body(ga_ref, whh_ref, lw_ref, out_ref):
        whh = whh_ref[...]

        def step(t, carry):
            h, c, acc = carry
            gt = ga_ref[t]
            ti = jnp.tanh(gt[0] + h * whh[0])
            tf = jnp.tanh(gt[1] + h * whh[1])
            tg = jnp.tanh(gt[2] + h * whh[2])
            to = jnp.tanh(gt[3] + h * whh[3])
            i_s = ti * 0.5 + 0.5
            f_s = tf * 0.5 + 0.5
            o_s = to * 0.5 + 0.5
            c = f_s * c + i_s * tg
            h = o_s * jnp.tanh(c)
            acc = acc + h * lw_ref[t]
            return h, c, acc

        z = jnp.zeros((8, 128), jnp.float32)
        _, _, acc = lax.fori_loop(0, T, step, (z, z, z))
        out_ref[...] = acc

    return body


@jax.jit
def _run(x, w16, b1, w2, b2, wih, whh, gbias, lw, lb):
    B, H, W = x.shape
    T = lw.shape[0]
    TP = ((T + 127) // 128) * 128
    bb = 16
    assert H == KH and W == 16 * T and B % bb == 0 and B % 512 == 0

    xpad = jnp.pad(x.astype(jnp.bfloat16),
                   ((0, 0), (0, 0), (1, 16 * (T + 1) - W - 1)))
    x16 = xpad.reshape(B, KH, T + 1, 16).transpose(0, 1, 3, 2)
    ones = jnp.ones((B, KH, 1, T), jnp.bfloat16)
    xph = jnp.concatenate(
        [x16[:, :, :, :T], x16[:, :, 0:2, 1:T + 1], ones], axis=2)
    xph = jnp.pad(xph, ((0, 0), (0, 0), (0, NPHP - NPH - 1), (0, TP - T)))

    w2c = w2.astype(jnp.bfloat16)
    w16 = jnp.concatenate(
        [w16, jnp.tile(b1, (16, 1))], axis=1).astype(jnp.bfloat16)
    w16 = jnp.pad(w16, ((0, 0), (0, NPHP - NPH - 1)))

    conv_body = _make_conv_kernel(bb, T, TP)
    pa = pl.pallas_call(
        conv_body,
        out_shape=jax.ShapeDtypeStruct((B, 8, TP), jnp.float32),
        grid=(B // bb,),
        in_specs=[
            pl.BlockSpec((bb, KH, NPHP, TP), lambda g: (g, 0, 0, 0)),
            pl.BlockSpec((16 * C1, NPHP), lambda g: (0, 0)),
            pl.BlockSpec((C2, 1), lambda g: (0, 0)),
            pl.BlockSpec((4 * C2, 6 * KH * C1), lambda g: (0, 0)),
            pl.BlockSpec((C1 // 2, 8), lambda g: (0, 0)),
            pl.BlockSpec((8, 1), lambda g: (0, 0)),
        ],
        out_specs=pl.BlockSpec((bb, 8, TP), lambda g: (g, 0, 0)),
        scratch_shapes=[pltpu.VMEM((6 * KH * C1, bb * TP), jnp.bfloat16),
                        pltpu.VMEM((C2, bb * TP), jnp.bfloat16)],
        compiler_params=pltpu.CompilerParams(
            dimension_semantics=("parallel",)),
    )(xph, w16, b2, w2c, wih, gbias.T)

    pat = pa[:, :, :T].reshape(4, 128, 2, 4, T)
    fwd = pat[:, :, 0]
    bwd = pat[:, :, 1][..., ::-1]
    ga = jnp.stack([fwd, bwd], axis=0)
    ga = ga.transpose(4, 3, 0, 1, 2).reshape(T, 4, 8, 128)

    whh_arr = jnp.repeat(whh, 4, axis=0).T.reshape(4, 8, 1)
    lw_arr = jnp.concatenate(
        [jnp.tile(lw[:, 0:1], (1, 4)), jnp.tile(lw[::-1, 1:2], (1, 4))],
        axis=1).reshape(T, 8, 1)

    scan_body = _make_scan_kernel(T)
    acc = pl.pallas_call(
        scan_body,
        out_shape=jax.ShapeDtypeStruct((8, 128), jnp.float32),
    )(ga, whh_arr, lw_arr)

    return (acc[0:4] + acc[4:8] + lb[0, 0]).reshape(B)


def kernel(x, w16, b1, w2, b2, wih, whh, gbias, lw, lb):
    return _run(x, w16, b1, w2, b2, wih, whh, gbias, lw, lb)


# speedup vs baseline: 1.0373x; 1.0373x over previous
import functools

import jax
import jax.numpy as jnp
from jax import lax
from jax.experimental import pallas as pl
from jax.experimental.pallas import tpu as pltpu

C1 = 64
C2 = 32
KH = 3
NPH = 18
NPHP = 32


def _make_conv_kernel(bb, T, TP):
    K1 = KH * C1

    def body(xph_ref, w16_ref, b2_ref, w2c_ref, wih_ref, gbt_ref,
             out_ref, h1_ref, feat_ref):
        lane = lax.broadcasted_iota(jnp.int32, (C1, TP), 1)
        b2 = b2_ref[...]
        gbt = gbt_ref[...]
        w2c = w2c_ref[...]
        wih = wih_ref[...]
        zcol = jnp.zeros((C1, 1), jnp.bfloat16)

        for b in range(bb):
            for kh in range(KH):
                ph = []
                for d in range(4):
                    rd = jnp.dot(w16_ref[256 * d:256 * d + 256, :],
                                 xph_ref[b, kh],
                                 preferred_element_type=jnp.float32)
                    pp = rd.reshape(4, C1, TP)
                    ph.append((jnp.maximum(jnp.maximum(pp[0], pp[1]),
                                           jnp.maximum(pp[2], pp[3]))
                               ).astype(jnp.bfloat16))
                p0 = jnp.where(lane < T, ph[0], 0.0)
                row = kh * C1
                col = b * TP
                h1_ref[1 * K1 + row:1 * K1 + row + C1, col:col + TP] = p0
                h1_ref[2 * K1 + row:2 * K1 + row + C1, col:col + TP] = ph[1]
                h1_ref[3 * K1 + row:3 * K1 + row + C1, col:col + TP] = ph[2]
                h1_ref[4 * K1 + row:4 * K1 + row + C1, col:col + TP] = ph[3]
                h1_ref[0 * K1 + row:0 * K1 + row + C1,
                       col:col + TP] = jnp.concatenate(
                    [zcol, ph[3][:, :TP - 1]], axis=1)
                h1_ref[5 * K1 + row:5 * K1 + row + C1,
                       col:col + TP] = jnp.concatenate(
                    [p0[:, 1:], zcol], axis=1)

        NC = 2 * TP
        for c in range(bb * TP // NC):
            s_c = jnp.dot(w2c, h1_ref[:, c * NC:(c + 1) * NC],
                          preferred_element_type=jnp.float32)
            sc = s_c.reshape(4, C2, NC)
            feat_ref[:, c * NC:(c + 1) * NC] = (
                jnp.maximum(jnp.maximum(sc[0], sc[1]),
                            jnp.maximum(sc[2], sc[3])) + b2)

        proj = lax.dot_general(
            wih, feat_ref[...], dimension_numbers=(((0,), (0,)), ((), ())),
            preferred_element_type=jnp.float32)
        for b in range(bb):
            out_ref[b] = proj[:, b * TP:(b + 1) * TP] + gbt

    return body


def _make_scan_kernel(T):
    def body(ga_ref, whh_ref, lw_ref, out_ref):
        whh = whh_ref[...]

        def step(t, carry):
            h, c, acc = carry
            gt = ga_ref[t]
            ti = jnp.tanh(gt[0] + h * whh[0])
            tf = jnp.tanh(gt[1] + h * whh[1])
            tg = jnp.tanh(gt[2] + h * whh[2])
            to = jnp.tanh(gt[3] + h * whh[3])
            i_s = ti * 0.5 + 0.5
            f_s = tf * 0.5 + 0.5
            o_s = to * 0.5 + 0.5
            c = f_s * c + i_s * tg
            h = o_s * jnp.tanh(c)
            acc = acc + h * lw_ref[t]
            return h, c, acc

        z = jnp.zeros((8, 128), jnp.float32)
        _, _, acc = lax.fori_loop(0, T, step, (z, z, z), unroll=5)
        out_ref[...] = acc

    return body


@jax.jit
def _run(x, w16, b1, w2, b2, wih, whh, gbias, lw, lb):
    B, H, W = x.shape
    T = lw.shape[0]
    TP = ((T + 127) // 128) * 128
    bb = 16
    assert H == KH and W == 16 * T and B % bb == 0 and B % 512 == 0

    xpad = jnp.pad(x.astype(jnp.bfloat16),
                   ((0, 0), (0, 0), (1, 16 * (T + 1) - W - 1)))
    x16 = xpad.reshape(B, KH, T + 1, 16).transpose(0, 1, 3, 2)
    ones = jnp.ones((B, KH, 1, T), jnp.bfloat16)
    xph = jnp.concatenate(
        [x16[:, :, :, :T], x16[:, :, 0:2, 1:T + 1], ones], axis=2)
    xph = jnp.pad(xph, ((0, 0), (0, 0), (0, NPHP - NPH - 1), (0, TP - T)))

    w2c = w2.astype(jnp.bfloat16)
    w16 = jnp.concatenate(
        [w16, jnp.tile(b1, (16, 1))], axis=1).astype(jnp.bfloat16)
    w16 = jnp.pad(w16, ((0, 0), (0, NPHP - NPH - 1)))

    conv_body = _make_conv_kernel(bb, T, TP)
    pa = pl.pallas_call(
        conv_body,
        out_shape=jax.ShapeDtypeStruct((B, 8, TP), jnp.float32),
        grid=(B // bb,),
        in_specs=[
            pl.BlockSpec((bb, KH, NPHP, TP), lambda g: (g, 0, 0, 0)),
            pl.BlockSpec((16 * C1, NPHP), lambda g: (0, 0)),
            pl.BlockSpec((C2, 1), lambda g: (0, 0)),
            pl.BlockSpec((4 * C2, 6 * KH * C1), lambda g: (0, 0)),
            pl.BlockSpec((C1 // 2, 8), lambda g: (0, 0)),
            pl.BlockSpec((8, 1), lambda g: (0, 0)),
        ],
        out_specs=pl.BlockSpec((bb, 8, TP), lambda g: (g, 0, 0)),
        scratch_shapes=[pltpu.VMEM((6 * KH * C1, bb * TP), jnp.bfloat16),
                        pltpu.VMEM((C2, bb * TP), jnp.float32)],
        compiler_params=pltpu.CompilerParams(
            dimension_semantics=("parallel",)),
    )(xph, w16, b2, w2c, wih, gbias.T)

    pat = pa[:, :, :T].reshape(4, 128, 2, 4, T)
    fwd = pat[:, :, 0]
    bwd = pat[:, :, 1][..., ::-1]
    ga = jnp.stack([fwd, bwd], axis=0)
    ga = ga.transpose(4, 3, 0, 1, 2).reshape(T, 4, 8, 128)

    whh_arr = jnp.repeat(whh, 4, axis=0).T.reshape(4, 8, 1)
    lw_arr = jnp.concatenate(
        [jnp.tile(lw[:, 0:1], (1, 4)), jnp.tile(lw[::-1, 1:2], (1, 4))],
        axis=1).reshape(T, 8, 1)

    scan_body = _make_scan_kernel(T)
    acc = pl.pallas_call(
        scan_body,
        out_shape=jax.ShapeDtypeStruct((8, 128), jnp.float32),
    )(ga, whh_arr, lw_arr)

    return (acc[0:4] + acc[4:8] + lb[0, 0]).reshape(B)


def kernel(x, w16, b1, w2, b2, wih, whh, gbias, lw, lb):
    return _run(x, w16, b1, w2, b2, wih, whh, gbias, lw, lb)


# speedup vs baseline: 1.0463x; 1.0087x over previous
import functools

import jax
import jax.numpy as jnp
from jax import lax
from jax.experimental import pallas as pl
from jax.experimental.pallas import tpu as pltpu

C1 = 64
C2 = 32
KH = 3
NPH = 18
NPHP = 32


def _make_conv_kernel(bb, T, TP):
    K1 = KH * C1

    def body(xph_ref, w16_ref, b2_ref, w2c_ref, wih_ref, gbt_ref,
             out_ref, h1_ref, feat_ref):
        lane = lax.broadcasted_iota(jnp.int32, (C1, TP), 1)
        b2 = b2_ref[...]
        gbt = gbt_ref[...]
        w2c = w2c_ref[...]
        wih = wih_ref[...]
        zcol = jnp.zeros((C1, 1), jnp.bfloat16)

        for b in range(bb):
            for kh in range(KH):
                ph = []
                for d in range(4):
                    rd = jnp.dot(w16_ref[256 * d:256 * d + 256, :],
                                 xph_ref[b, kh],
                                 preferred_element_type=jnp.float32)
                    pp = rd.reshape(4, C1, TP)
                    ph.append((jnp.maximum(jnp.maximum(pp[0], pp[1]),
                                           jnp.maximum(pp[2], pp[3]))
                               ).astype(jnp.bfloat16))
                p0 = jnp.where(lane < T, ph[0], 0.0)
                row = kh * C1
                col = b * TP
                h1_ref[1 * K1 + row:1 * K1 + row + C1, col:col + TP] = p0
                h1_ref[2 * K1 + row:2 * K1 + row + C1, col:col + TP] = ph[1]
                h1_ref[3 * K1 + row:3 * K1 + row + C1, col:col + TP] = ph[2]
                h1_ref[4 * K1 + row:4 * K1 + row + C1, col:col + TP] = ph[3]
                h1_ref[0 * K1 + row:0 * K1 + row + C1,
                       col:col + TP] = jnp.concatenate(
                    [zcol, ph[3][:, :TP - 1]], axis=1)
                h1_ref[5 * K1 + row:5 * K1 + row + C1,
                       col:col + TP] = jnp.concatenate(
                    [p0[:, 1:], zcol], axis=1)

        NC = 2 * TP
        for c in range(bb * TP // NC):
            s_c = jnp.dot(w2c, h1_ref[:, c * NC:(c + 1) * NC],
                          preferred_element_type=jnp.float32)
            sc = s_c.reshape(4, C2, NC)
            feat_ref[:, c * NC:(c + 1) * NC] = (
                jnp.maximum(jnp.maximum(sc[0], sc[1]),
                            jnp.maximum(sc[2], sc[3])) + b2)

        proj = lax.dot_general(
            wih, feat_ref[...], dimension_numbers=(((0,), (0,)), ((), ())),
            preferred_element_type=jnp.float32)
        for b in range(bb):
            out_ref[b] = proj[:, b * TP:(b + 1) * TP] + gbt

    return body


def _make_scan_kernel(T):
    def body(ga_ref, whh_ref, lw_ref, out_ref):
        whh = whh_ref[...]

        def step(t, carry):
            h, c, acc = carry
            gt = ga_ref[t]
            ti = jnp.tanh(gt[0] + h * whh[0])
            tf = jnp.tanh(gt[1] + h * whh[1])
            tg = jnp.tanh(gt[2] + h * whh[2])
            to = jnp.tanh(gt[3] + h * whh[3])
            i_s = ti * 0.5 + 0.5
            f_s = tf * 0.5 + 0.5
            o_s = to * 0.5 + 0.5
            c = f_s * c + i_s * tg
            h = o_s * jnp.tanh(c)
            acc = acc + h * lw_ref[t]
            return h, c, acc

        z = jnp.zeros((8, 128), jnp.float32)
        _, _, acc = lax.fori_loop(0, T, step, (z, z, z), unroll=15)
        out_ref[...] = acc

    return body


@jax.jit
def _run(x, w16, b1, w2, b2, wih, whh, gbias, lw, lb):
    B, H, W = x.shape
    T = lw.shape[0]
    TP = ((T + 127) // 128) * 128
    bb = 16
    assert H == KH and W == 16 * T and B % bb == 0 and B % 512 == 0

    xpad = jnp.pad(x.astype(jnp.bfloat16),
                   ((0, 0), (0, 0), (1, 16 * (T + 1) - W - 1)))
    x16 = xpad.reshape(B, KH, T + 1, 16).transpose(0, 1, 3, 2)
    ones = jnp.ones((B, KH, 1, T), jnp.bfloat16)
    xph = jnp.concatenate(
        [x16[:, :, :, :T], x16[:, :, 0:2, 1:T + 1], ones], axis=2)
    xph = jnp.pad(xph, ((0, 0), (0, 0), (0, NPHP - NPH - 1), (0, TP - T)))

    w2c = w2.astype(jnp.bfloat16)
    w16 = jnp.concatenate(
        [w16, jnp.tile(b1, (16, 1))], axis=1).astype(jnp.bfloat16)
    w16 = jnp.pad(w16, ((0, 0), (0, NPHP - NPH - 1)))

    conv_body = _make_conv_kernel(bb, T, TP)
    pa = pl.pallas_call(
        conv_body,
        out_shape=jax.ShapeDtypeStruct((B, 8, TP), jnp.float32),
        grid=(B // bb,),
        in_specs=[
            pl.BlockSpec((bb, KH, NPHP, TP), lambda g: (g, 0, 0, 0)),
            pl.BlockSpec((16 * C1, NPHP), lambda g: (0, 0)),
            pl.BlockSpec((C2, 1), lambda g: (0, 0)),
            pl.BlockSpec((4 * C2, 6 * KH * C1), lambda g: (0, 0)),
            pl.BlockSpec((C1 // 2, 8), lambda g: (0, 0)),
            pl.BlockSpec((8, 1), lambda g: (0, 0)),
        ],
        out_specs=pl.BlockSpec((bb, 8, TP), lambda g: (g, 0, 0)),
        scratch_shapes=[pltpu.VMEM((6 * KH * C1, bb * TP), jnp.bfloat16),
                        pltpu.VMEM((C2, bb * TP), jnp.float32)],
        compiler_params=pltpu.CompilerParams(
            dimension_semantics=("parallel",)),
    )(xph, w16, b2, w2c, wih, gbias.T)

    pat = pa[:, :, :T].reshape(4, 128, 2, 4, T)
    fwd = pat[:, :, 0]
    bwd = pat[:, :, 1][..., ::-1]
    ga = jnp.stack([fwd, bwd], axis=0)
    ga = ga.transpose(4, 3, 0, 1, 2).reshape(T, 4, 8, 128)

    whh_arr = jnp.repeat(whh, 4, axis=0).T.reshape(4, 8, 1)
    lw_arr = jnp.concatenate(
        [jnp.tile(lw[:, 0:1], (1, 4)), jnp.tile(lw[::-1, 1:2], (1, 4))],
        axis=1).reshape(T, 8, 1)

    scan_body = _make_scan_kernel(T)
    acc = pl.pallas_call(
        scan_body,
        out_shape=jax.ShapeDtypeStruct((8, 128), jnp.float32),
    )(ga, whh_arr, lw_arr)

    return (acc[0:4] + acc[4:8] + lb[0, 0]).reshape(B)


def kernel(x, w16, b1, w2, b2, wih, whh, gbias, lw, lb):
    return _run(x, w16, b1, w2, b2, wih, whh, gbias, lw, lb)


# speedup vs baseline: 1.0502x; 1.0037x over previous
import functools

import jax
import jax.numpy as jnp
from jax import lax
from jax.experimental import pallas as pl
from jax.experimental.pallas import tpu as pltpu

C1 = 64
C2 = 32
KH = 3
NPH = 18
NPHP = 32


def _make_conv_kernel(bb, T, TP):
    K1 = KH * C1

    def body(xph_ref, w16_ref, b2_ref, w2c_ref, wih_ref, gbt_ref,
             out_ref, h1_ref, feat_ref):
        lane = lax.broadcasted_iota(jnp.int32, (C1, TP), 1)
        b2 = b2_ref[...]
        gbt = gbt_ref[...]
        w2c = w2c_ref[...]
        wih = wih_ref[...]
        zcol = jnp.zeros((C1, 1), jnp.bfloat16)

        for b in range(bb):
            for kh in range(KH):
                ph = []
                for d in range(4):
                    rd = jnp.dot(w16_ref[256 * d:256 * d + 256, :],
                                 xph_ref[b, kh],
                                 preferred_element_type=jnp.float32)
                    pp = rd.reshape(4, C1, TP)
                    ph.append((jnp.maximum(jnp.maximum(pp[0], pp[1]),
                                           jnp.maximum(pp[2], pp[3]))
                               ).astype(jnp.bfloat16))
                p0 = jnp.where(lane < T, ph[0], 0.0)
                row = kh * C1
                col = b * TP
                h1_ref[1 * K1 + row:1 * K1 + row + C1, col:col + TP] = p0
                h1_ref[2 * K1 + row:2 * K1 + row + C1, col:col + TP] = ph[1]
                h1_ref[3 * K1 + row:3 * K1 + row + C1, col:col + TP] = ph[2]
                h1_ref[4 * K1 + row:4 * K1 + row + C1, col:col + TP] = ph[3]
                h1_ref[0 * K1 + row:0 * K1 + row + C1,
                       col:col + TP] = jnp.concatenate(
                    [zcol, ph[3][:, :TP - 1]], axis=1)
                h1_ref[5 * K1 + row:5 * K1 + row + C1,
                       col:col + TP] = jnp.concatenate(
                    [p0[:, 1:], zcol], axis=1)

        NC = 2 * TP
        for c in range(bb * TP // NC):
            s_c = jnp.dot(w2c, h1_ref[:, c * NC:(c + 1) * NC],
                          preferred_element_type=jnp.float32)
            sc = s_c.reshape(4, C2, NC)
            feat_ref[:, c * NC:(c + 1) * NC] = (
                jnp.maximum(jnp.maximum(sc[0], sc[1]),
                            jnp.maximum(sc[2], sc[3])) + b2)

        proj = lax.dot_general(
            wih, feat_ref[...], dimension_numbers=(((0,), (0,)), ((), ())),
            preferred_element_type=jnp.float32)
        for b in range(bb):
            out_ref[b] = proj[:, b * TP:(b + 1) * TP] + gbt

    return body


def _make_scan_kernel(T):
    def body(ga_ref, whh_ref, lw_ref, out_ref):
        whh = whh_ref[...]

        def step(t, carry):
            h, c, acc = carry
            gt = ga_ref[t]
            ti = jnp.tanh(gt[0] + h * whh[0])
            tf = jnp.tanh(gt[1] + h * whh[1])
            tg = jnp.tanh(gt[2] + h * whh[2])
            to = jnp.tanh(gt[3] + h * whh[3])
            i_s = ti * 0.5 + 0.5
            f_s = tf * 0.5 + 0.5
            o_s = to * 0.5 + 0.5
            c = f_s * c + i_s * tg
            h = o_s * jnp.tanh(c)
            acc = acc + h * lw_ref[t]
            return h, c, acc

        z = jnp.zeros((8, 128), jnp.float32)
        _, _, acc = lax.fori_loop(0, T, step, (z, z, z), unroll=15)
        out_ref[...] = acc

    return body


@jax.jit
def _run(x, w16, b1, w2, b2, wih, whh, gbias, lw, lb):
    B, H, W = x.shape
    T = lw.shape[0]
    TP = ((T + 127) // 128) * 128
    bb = 32
    assert H == KH and W == 16 * T and B % bb == 0 and B % 512 == 0

    xpad = jnp.pad(x.astype(jnp.bfloat16),
                   ((0, 0), (0, 0), (1, 16 * (T + 1) - W - 1)))
    x16 = xpad.reshape(B, KH, T + 1, 16).transpose(0, 1, 3, 2)
    ones = jnp.ones((B, KH, 1, T), jnp.bfloat16)
    xph = jnp.concatenate(
        [x16[:, :, :, :T], x16[:, :, 0:2, 1:T + 1], ones], axis=2)
    xph = jnp.pad(xph, ((0, 0), (0, 0), (0, NPHP - NPH - 1), (0, TP - T)))

    w2c = w2.astype(jnp.bfloat16)
    w16 = jnp.concatenate(
        [w16, jnp.tile(b1, (16, 1))], axis=1).astype(jnp.bfloat16)
    w16 = jnp.pad(w16, ((0, 0), (0, NPHP - NPH - 1)))

    conv_body = _make_conv_kernel(bb, T, TP)
    pa = pl.pallas_call(
        conv_body,
        out_shape=jax.ShapeDtypeStruct((B, 8, TP), jnp.float32),
        grid=(B // bb,),
        in_specs=[
            pl.BlockSpec((bb, KH, NPHP, TP), lambda g: (g, 0, 0, 0)),
            pl.BlockSpec((16 * C1, NPHP), lambda g: (0, 0)),
            pl.BlockSpec((C2, 1), lambda g: (0, 0)),
            pl.BlockSpec((4 * C2, 6 * KH * C1), lambda g: (0, 0)),
            pl.BlockSpec((C1 // 2, 8), lambda g: (0, 0)),
            pl.BlockSpec((8, 1), lambda g: (0, 0)),
        ],
        out_specs=pl.BlockSpec((bb, 8, TP), lambda g: (g, 0, 0)),
        scratch_shapes=[pltpu.VMEM((6 * KH * C1, bb * TP), jnp.bfloat16),
                        pltpu.VMEM((C2, bb * TP), jnp.float32)],
        compiler_params=pltpu.CompilerParams(
            dimension_semantics=("parallel",)),
    )(xph, w16, b2, w2c, wih, gbias.T)

    pat = pa[:, :, :T].reshape(4, 128, 2, 4, T)
    fwd = pat[:, :, 0]
    bwd = pat[:, :, 1][..., ::-1]
    ga = jnp.stack([fwd, bwd], axis=0)
    ga = ga.transpose(4, 3, 0, 1, 2).reshape(T, 4, 8, 128)

    whh_arr = jnp.repeat(whh, 4, axis=0).T.reshape(4, 8, 1)
    lw_arr = jnp.concatenate(
        [jnp.tile(lw[:, 0:1], (1, 4)), jnp.tile(lw[::-1, 1:2], (1, 4))],
        axis=1).reshape(T, 8, 1)

    scan_body = _make_scan_kernel(T)
    acc = pl.pallas_call(
        scan_body,
        out_shape=jax.ShapeDtypeStruct((8, 128), jnp.float32),
    )(ga, whh_arr, lw_arr)

    return (acc[0:4] + acc[4:8] + lb[0, 0]).reshape(B)


def kernel(x, w16, b1, w2, b2, wih, whh, gbias, lw, lb):
    return _run(x, w16, b1, w2, b2, wih, whh, gbias, lw, lb)


# speedup vs baseline: 1.0506x; 1.0004x over previous
import jax
import jax.numpy as jnp
from jax import lax
from jax.experimental import pallas as pl
from jax.experimental.pallas import tpu as pltpu

C1 = 64
C2 = 32
KH = 3
NPH = 18
NPHP = 32


def _make_conv_kernel(bb, T, TP):
    K1 = KH * C1

    def body(xph_ref, w16_ref, b2_ref, w2c_ref, wih_ref, gbt_ref,
             out_ref, h1_ref, feat_ref):
        lane = lax.broadcasted_iota(jnp.int32, (C1, TP), 1)
        b2 = b2_ref[...]
        gbt = gbt_ref[...]
        w2c = w2c_ref[...]
        wih = wih_ref[...]
        zcol = jnp.zeros((C1, 1), jnp.bfloat16)

        for b in range(bb):
            for kh in range(KH):
                ph = []
                for d in range(4):
                    rd = jnp.dot(w16_ref[256 * d:256 * d + 256, :],
                                 xph_ref[b, kh],
                                 preferred_element_type=jnp.float32)
                    pp = rd.reshape(4, C1, TP)
                    ph.append((jnp.maximum(jnp.maximum(pp[0], pp[1]),
                                           jnp.maximum(pp[2], pp[3]))
                               ).astype(jnp.bfloat16))
                p0 = jnp.where(lane < T, ph[0], 0.0)
                row = kh * C1
                col = b * TP
                h1_ref[1 * K1 + row:1 * K1 + row + C1, col:col + TP] = p0
                h1_ref[2 * K1 + row:2 * K1 + row + C1, col:col + TP] = ph[1]
                h1_ref[3 * K1 + row:3 * K1 + row + C1, col:col + TP] = ph[2]
                h1_ref[4 * K1 + row:4 * K1 + row + C1, col:col + TP] = ph[3]
                h1_ref[0 * K1 + row:0 * K1 + row + C1,
                       col:col + TP] = jnp.concatenate(
                    [zcol, ph[3][:, :TP - 1]], axis=1)
                h1_ref[5 * K1 + row:5 * K1 + row + C1,
                       col:col + TP] = jnp.concatenate(
                    [p0[:, 1:], zcol], axis=1)

        NC = 2 * TP
        for c in range(bb * TP // NC):
            s_c = jnp.dot(w2c, h1_ref[:, c * NC:(c + 1) * NC],
                          preferred_element_type=jnp.float32)
            sc = s_c.reshape(4, C2, NC)
            feat_ref[:, c * NC:(c + 1) * NC] = (
                jnp.maximum(jnp.maximum(sc[0], sc[1]),
                            jnp.maximum(sc[2], sc[3])) + b2)

        proj = lax.dot_general(
            wih, feat_ref[...], dimension_numbers=(((0,), (0,)), ((), ())),
            preferred_element_type=jnp.float32)
        for b in range(bb):
            out_ref[b] = proj[:, b * TP:(b + 1) * TP] + gbt

    return body


def _make_scan_kernel(T):
    def body(ga_ref, whh_ref, lw_ref, out_ref):
        whh = whh_ref[...]

        def step(t, carry):
            h, c, acc = carry
            gt = ga_ref[t]
            ti = jnp.tanh(gt[0] + h * whh[0])
            tf = jnp.tanh(gt[1] + h * whh[1])
            tg = jnp.tanh(gt[2] + h * whh[2])
            to = jnp.tanh(gt[3] + h * whh[3])
            i_s = ti * 0.5 + 0.5
            f_s = tf * 0.5 + 0.5
            o_s = to * 0.5 + 0.5
            c = f_s * c + i_s * tg
            h = o_s * jnp.tanh(c)
            acc = acc + h * lw_ref[t]
            return h, c, acc

        z = jnp.zeros((8, 128), jnp.float32)
        _, _, acc = lax.fori_loop(0, T, step, (z, z, z), unroll=15)
        out_ref[...] = acc

    return body


@jax.jit
def _run(x, w16, b1, w2, b2, wih, whh, gbias, lw, lb):
    B, H, W = x.shape
    T = lw.shape[0]
    TP = ((T + 127) // 128) * 128
    bb = 32
    assert H == KH and W == 16 * T and B % bb == 0 and B % 512 == 0

    xpad = jnp.pad(x.astype(jnp.bfloat16),
                   ((0, 0), (0, 0), (1, 16 * (T + 1) - W - 1)))
    x16 = xpad.reshape(B, KH, T + 1, 16).transpose(0, 1, 3, 2)
    ones = jnp.ones((B, KH, 1, T), jnp.bfloat16)
    xph = jnp.concatenate(
        [x16[:, :, :, :T], x16[:, :, 0:2, 1:T + 1], ones], axis=2)
    xph = jnp.pad(xph, ((0, 0), (0, 0), (0, NPHP - NPH - 1), (0, TP - T)))

    w2c = w2.astype(jnp.bfloat16)
    w16 = jnp.concatenate(
        [w16, jnp.tile(b1, (16, 1))], axis=1).astype(jnp.bfloat16)
    w16 = jnp.pad(w16, ((0, 0), (0, NPHP - NPH - 1)))

    conv_body = _make_conv_kernel(bb, T, TP)
    pa = pl.pallas_call(
        conv_body,
        out_shape=jax.ShapeDtypeStruct((B, 8, TP), jnp.float32),
        grid=(B // bb,),
        in_specs=[
            pl.BlockSpec((bb, KH, NPHP, TP), lambda g: (g, 0, 0, 0)),
            pl.BlockSpec((16 * C1, NPHP), lambda g: (0, 0)),
            pl.BlockSpec((C2, 1), lambda g: (0, 0)),
            pl.BlockSpec((4 * C2, 6 * KH * C1), lambda g: (0, 0)),
            pl.BlockSpec((C1 // 2, 8), lambda g: (0, 0)),
            pl.BlockSpec((8, 1), lambda g: (0, 0)),
        ],
        out_specs=pl.BlockSpec((bb, 8, TP), lambda g: (g, 0, 0)),
        scratch_shapes=[pltpu.VMEM((6 * KH * C1, bb * TP), jnp.bfloat16),
                        pltpu.VMEM((C2, bb * TP), jnp.float32)],
        compiler_params=pltpu.CompilerParams(
            dimension_semantics=("parallel",)),
    )(xph, w16, b2, w2c, wih, gbias.T)

    pat = pa[:, :, :T].reshape(4, 128, 2, 4, T)
    fwd = pat[:, :, 0]
    bwd = pat[:, :, 1][..., ::-1]
    ga = jnp.stack([fwd, bwd], axis=0)
    ga = ga.transpose(4, 3, 0, 1, 2).reshape(T, 4, 8, 128)

    whh_arr = jnp.repeat(whh, 4, axis=0).T.reshape(4, 8, 1)
    lw_arr = jnp.concatenate(
        [jnp.tile(lw[:, 0:1], (1, 4)), jnp.tile(lw[::-1, 1:2], (1, 4))],
        axis=1).reshape(T, 8, 1)

    scan_body = _make_scan_kernel(T)
    acc = pl.pallas_call(
        scan_body,
        out_shape=jax.ShapeDtypeStruct((8, 128), jnp.float32),
    )(ga, whh_arr, lw_arr)

    return (acc[0:4] + acc[4:8] + lb[0, 0]).reshape(B)


def kernel(x, w16, b1, w2, b2, wih, whh, gbias, lw, lb):
    return _run(x, w16, b1, w2, b2, wih, whh, gbias, lw, lb)
